# Initial kernel scaffold; baseline (speedup 1.0000x reference)
import jax
import jax.numpy as jnp
from jax.experimental import pallas as pl


def kernel(predicts_t, feature_v, decision_words, embed_words, W_fuse, b_fuse, W_cls, b_cls):
    raise NotImplementedError("write your pallas kernel here")


# speedup vs baseline: 3.7597x; 3.7597x over previous
import functools

import jax
import jax.numpy as jnp
from jax import lax
from jax.experimental import pallas as pl
from jax.experimental.pallas import tpu as pltpu
from jax.experimental.pallas import tpu_sc as plsc

B = 64
T = 8
V = 1000
D = 128
K = 32
L = 50
NUM_CLS = 200
BETA_KNOW = 0.5
BETA_REL = 0.2

NC = 2
NS = 16
LANE = 16
NW = NC * NS
ROWS_PER_W = B // NW


def _score_body(pt_ref, dw_ref, wf_ref, ids_ref, coef_ref):
    iota = lax.broadcasted_iota(jnp.int32, (B, V), 1)
    model_pre = None
    for t in range(T):
        x = pt_ref[:, t, :]
        mn = jnp.min(x, axis=1, keepdims=True)
        mx = jnp.max(x, axis=1, keepdims=True)
        nrm = (x - mn) / (mx - mn)
        model_pre = nrm if model_pre is None else jnp.maximum(model_pre, nrm)
    e = jnp.exp(model_pre - jnp.max(model_pre, axis=1, keepdims=True))
    sm_m = e / jnp.sum(e, axis=1, keepdims=True)
    cnt = jnp.zeros((B, V), jnp.float32)
    for l in range(L):
        cnt = cnt + jnp.where(iota == dw_ref[:, l : l + 1], 1.0, 0.0)
    ec = jnp.exp(cnt - jnp.max(cnt, axis=1, keepdims=True))
    sm_c = ec / jnp.sum(ec, axis=1, keepdims=True)
    refine = (1.0 - BETA_KNOW) * sm_m + BETA_KNOW * sm_c
    r = refine
    cols = []
    for _ in range(K):
        mval = jnp.max(r, axis=1, keepdims=True)
        idx = jnp.min(jnp.where(r == mval, iota, jnp.int32(V)), axis=1, keepdims=True)
        cols.append(idx)
        r = jnp.where(iota == idx, -1.0, r)
    ids = jnp.concatenate(cols, axis=1)
    m = jnp.zeros((B, K), jnp.float32)
    for l in range(L):
        m = jnp.maximum(m, jnp.where(ids == dw_ref[:, l : l + 1], 1.0, 0.0))
    s = jnp.sum(m, axis=1, keepdims=True)
    deg = (1.0 + BETA_REL * (K - 1)) + BETA_REL * s * m
    wd = wf_ref[...] / deg
    s0 = jnp.sum(wd, axis=1, keepdims=True)
    s1 = jnp.sum(wd * m, axis=1, keepdims=True)
    coef = BETA_REL * s0 + (1.0 - BETA_REL) * wd + BETA_REL * s1 * m
    ids_ref[...] = ids
    coef_ref[...] = coef


def _bag_body(ids_hbm, cb_hbm, emb_hbm, out_hbm, idx_v, cb_v, rows_v, out_v, sem):
    wid = lax.axis_index("s") * NC + lax.axis_index("c")
    for rr in range(ROWS_PER_W):
        bb = wid * ROWS_PER_W + rr
        pltpu.sync_copy(ids_hbm.at[bb], idx_v)
        pltpu.sync_copy(cb_hbm.at[bb], cb_v)
        pltpu.async_copy(emb_hbm.at[idx_v], rows_v, sem).wait()
        acc = [jnp.zeros((LANE,), jnp.float32) for _ in range(D // LANE)]
        for i in range(K):
            cvec = cb_v[i, :]
            for c in range(D // LANE):
                acc[c] = acc[c] + cvec * rows_v[i, pl.ds(c * LANE, LANE)]
        for c in range(D // LANE):
            out_v[pl.ds(c * LANE, LANE)] = acc[c]
        pltpu.sync_copy(out_v, out_hbm.at[bb])


def _bag_call(ids, coef_b, emb):
    fn = functools.partial(
        pl.kernel,
        mesh=plsc.VectorSubcoreMesh(core_axis_name="c", subcore_axis_name="s"),
        out_type=jax.ShapeDtypeStruct((B, D), jnp.float32),
        scratch_types=[
            pltpu.VMEM((K,), jnp.int32),
            pltpu.VMEM((K, LANE), jnp.float32),
            pltpu.VMEM((K, D), jnp.float32),
            pltpu.VMEM((D,), jnp.float32),
            pltpu.SemaphoreType.DMA,
        ],
    )(_bag_body)
    return fn(ids, coef_b, emb)


def _head_body(word_ref, fv_ref, coef_ref, bf_ref, wct_ref, bc_ref, mix_ref, pv_ref):
    csum = jnp.sum(coef_ref[...], axis=1, keepdims=True)
    bf = bf_ref[0, 0]
    mixw = word_ref[...] + bf
    mixv = fv_ref[...] * csum + bf
    mix = jnp.concatenate([mixw, mixv], axis=1)
    mix_ref[...] = mix
    pv = lax.dot_general(
        mix, wct_ref[...], (((1,), (0,)), ((), ())),
        preferred_element_type=jnp.float32,
    )
    pv_ref[...] = pv + bc_ref[...]


def kernel(predicts_t, feature_v, decision_words, embed_words, W_fuse, b_fuse, W_cls, b_cls):
    dw = decision_words.astype(jnp.int32)
    ids, coef = pl.pallas_call(
        _score_body,
        out_shape=[
            jax.ShapeDtypeStruct((B, K), jnp.int32),
            jax.ShapeDtypeStruct((B, K), jnp.float32),
        ],
    )(predicts_t, dw, W_fuse)
    coef_b = jnp.broadcast_to(coef[:, :, None], (B, K, LANE))
    word = _bag_call(ids, coef_b, embed_words)
    mix, pv = pl.pallas_call(
        _head_body,
        out_shape=[
            jax.ShapeDtypeStruct((B, 2 * D), jnp.float32),
            jax.ShapeDtypeStruct((B, NUM_CLS), jnp.float32),
        ],
    )(word, feature_v, coef, b_fuse.reshape(1, 1), W_cls.T, b_cls.reshape(1, NUM_CLS))
    return (mix, pv)


# speedup vs baseline: 3.8163x; 1.0150x over previous
import functools

import jax
import jax.numpy as jnp
from jax import lax
from jax.experimental import pallas as pl
from jax.experimental.pallas import tpu as pltpu
from jax.experimental.pallas import tpu_sc as plsc

B = 64
T = 8
V = 1000
D = 128
K = 32
L = 50
NUM_CLS = 200
BETA_KNOW = 0.5
BETA_REL = 0.2

NC = 2
NS = 16
LANE = 16
NW = NC * NS
ROWS_PER_W = B // NW


def _score_body(pt_ref, dw_ref, wf_ref, ids_ref, coef_ref):
    iota = lax.broadcasted_iota(jnp.int32, (B, V), 1)
    model_pre = None
    for t in range(T):
        x = pt_ref[:, t, :]
        mn = jnp.min(x, axis=1, keepdims=True)
        mx = jnp.max(x, axis=1, keepdims=True)
        nrm = (x - mn) * (1.0 / (mx - mn))
        model_pre = nrm if model_pre is None else jnp.maximum(model_pre, nrm)
    e = jnp.exp(model_pre - jnp.max(model_pre, axis=1, keepdims=True))
    sm_m = e * (1.0 / jnp.sum(e, axis=1, keepdims=True))
    cnt = jnp.zeros((B, V), jnp.float32)
    for l in range(L):
        cnt = cnt + jnp.where(iota == dw_ref[:, l : l + 1], 1.0, 0.0)
    ec = jnp.exp(cnt - jnp.max(cnt, axis=1, keepdims=True))
    sm_c = ec * (1.0 / jnp.sum(ec, axis=1, keepdims=True))
    refine = (1.0 - BETA_KNOW) * sm_m + BETA_KNOW * sm_c
    r = refine
    cols = []
    for _ in range(K):
        mval = jnp.max(r, axis=1, keepdims=True)
        idx = jnp.min(jnp.where(r == mval, iota, jnp.int32(V)), axis=1, keepdims=True)
        cols.append(idx)
        r = jnp.where(iota == idx, -1.0, r)
    ids = jnp.concatenate(cols, axis=1)
    m = jnp.zeros((B, K), jnp.float32)
    for l in range(L):
        m = jnp.maximum(m, jnp.where(ids == dw_ref[:, l : l + 1], 1.0, 0.0))
    s = jnp.sum(m, axis=1, keepdims=True)
    deg = (1.0 + BETA_REL * (K - 1)) + BETA_REL * s * m
    wd = wf_ref[...] / deg
    s0 = jnp.sum(wd, axis=1, keepdims=True)
    s1 = jnp.sum(wd * m, axis=1, keepdims=True)
    coef = BETA_REL * s0 + (1.0 - BETA_REL) * wd + BETA_REL * s1 * m
    ids_ref[...] = ids
    coef_ref[...] = coef


KW = K * ROWS_PER_W


def _bag_body(ids_hbm, cb_hbm, emb_hbm, out_hbm, idx_v, cb_v, rows_v, out_v, sem):
    wid = lax.axis_index("s") * NC + lax.axis_index("c")
    pltpu.sync_copy(ids_hbm.at[pl.ds(wid * KW, KW)], idx_v)
    gather = pltpu.async_copy(emb_hbm.at[idx_v], rows_v, sem)
    pltpu.sync_copy(cb_hbm.at[pl.ds(wid * KW, KW)], cb_v)
    gather.wait()
    for rr in range(ROWS_PER_W):
        acc = [jnp.zeros((LANE,), jnp.float32) for _ in range(D // LANE)]
        for i in range(K):
            cvec = cb_v[rr * K + i, :]
            for c in range(D // LANE):
                acc[c] = acc[c] + cvec * rows_v[rr * K + i, pl.ds(c * LANE, LANE)]
        for c in range(D // LANE):
            out_v[rr, pl.ds(c * LANE, LANE)] = acc[c]
    pltpu.sync_copy(out_v, out_hbm.at[pl.ds(wid * ROWS_PER_W, ROWS_PER_W)])


def _bag_call(ids, coef_b, emb):
    fn = functools.partial(
        pl.kernel,
        mesh=plsc.VectorSubcoreMesh(core_axis_name="c", subcore_axis_name="s"),
        out_type=jax.ShapeDtypeStruct((B, D), jnp.float32),
        scratch_types=[
            pltpu.VMEM((KW,), jnp.int32),
            pltpu.VMEM((KW, LANE), jnp.float32),
            pltpu.VMEM((KW, D), jnp.float32),
            pltpu.VMEM((ROWS_PER_W, D), jnp.float32),
            pltpu.SemaphoreType.DMA,
        ],
    )(_bag_body)
    return fn(ids.reshape(B * K), coef_b.reshape(B * K, LANE), emb)


def _head_body(word_ref, fv_ref, coef_ref, bf_ref, wct_ref, bc_ref, mix_ref, pv_ref):
    csum = jnp.sum(coef_ref[...], axis=1, keepdims=True)
    bf = bf_ref[0, 0]
    mixw = word_ref[...] + bf
    mixv = fv_ref[...] * csum + bf
    mix = jnp.concatenate([mixw, mixv], axis=1)
    mix_ref[...] = mix
    pv = lax.dot_general(
        mix, wct_ref[...], (((1,), (1,)), ((), ())),
        preferred_element_type=jnp.float32,
    )
    pv_ref[...] = pv + bc_ref[...]


def kernel(predicts_t, feature_v, decision_words, embed_words, W_fuse, b_fuse, W_cls, b_cls):
    dw = decision_words.astype(jnp.int32)
    ids, coef = pl.pallas_call(
        _score_body,
        out_shape=[
            jax.ShapeDtypeStruct((B, K), jnp.int32),
            jax.ShapeDtypeStruct((B, K), jnp.float32),
        ],
    )(predicts_t, dw, W_fuse)
    coef_b = jnp.broadcast_to(coef[:, :, None], (B, K, LANE))
    word = _bag_call(ids, coef_b, embed_words)
    mix, pv = pl.pallas_call(
        _head_body,
        out_shape=[
            jax.ShapeDtypeStruct((B, 2 * D), jnp.float32),
            jax.ShapeDtypeStruct((B, NUM_CLS), jnp.float32),
        ],
    )(word, feature_v, coef, b_fuse.reshape(1, 1), W_cls, b_cls.reshape(1, NUM_CLS))
    return (mix, pv)
